# baseline (speedup 1.0000x reference)
import functools

import jax
import jax.numpy as jnp
from jax import lax
from jax.experimental import pallas as pl
from jax.experimental.pallas import tpu as pltpu
from jax.experimental.pallas import tpu_sc as plsc

_EPS = 1e-12
_L = 16
_NC = 2
_NS = 16
_NW = _NC * _NS
_R = 64


def _sc_call(x_c, x_delta, wb, bb, wo, bo):
    n, d = x_c.shape
    rows_per_w = n // _NW
    chunks = rows_per_w // _R
    jvec = d // _L

    mesh = plsc.VectorSubcoreMesh(core_axis_name="c", subcore_axis_name="s")

    @functools.partial(
        pl.kernel,
        mesh=mesh,
        out_type=[
            jax.ShapeDtypeStruct((n, d), jnp.float32),
            jax.ShapeDtypeStruct((n, d), jnp.float32),
            jax.ShapeDtypeStruct((n,), jnp.float32),
        ],
        scratch_types=[
            pltpu.VMEM((_R, d), jnp.float32),
            pltpu.VMEM((_R, d), jnp.float32),
            pltpu.VMEM((_R, d), jnp.float32),
            pltpu.VMEM((_R, d), jnp.float32),
            pltpu.VMEM((d,), jnp.float32),
            pltpu.VMEM((d,), jnp.float32),
            pltpu.VMEM((d,), jnp.float32),
            pltpu.VMEM((d,), jnp.float32),
            pltpu.VMEM((d,), jnp.float32),
            pltpu.VMEM((_R,), jnp.float32),
        ],
    )
    def k(xc_hbm, xd_hbm, wb_hbm, bb_hbm, wo_hbm, bo_hbm,
          oc_hbm, od_hbm, oa_hbm,
          xc_v, xd_v, oc_v, od_v, wb_v, bb_v, wo_v, bo_v, awb_v, oa_v):
        wid = lax.axis_index("s") * _NC + lax.axis_index("c")
        base = wid * rows_per_w

        pltpu.sync_copy(wb_hbm, wb_v)
        pltpu.sync_copy(bb_hbm, bb_v)
        pltpu.sync_copy(wo_hbm, wo_v)
        pltpu.sync_copy(bo_hbm, bo_v)
        for j in range(jvec):
            sl = pl.ds(j * _L, _L)
            awb_v[sl] = jnp.abs(wb_v[sl])
        head = pl.ds(0, _L)
        wb0 = wb_v[head][0]
        bb0 = bb_v[head][0]
        wo0 = wo_v[head][0]
        bo0 = bo_v[head][0]
        awb0 = awb_v[head][0]
        lane = lax.iota(jnp.int32, _L)
        mask0 = lane == 0

        def do_chunk(g, carry):
            cb = base + g * _R
            pltpu.sync_copy(xc_hbm.at[pl.ds(cb, _R)], xc_v)
            pltpu.sync_copy(xd_hbm.at[pl.ds(cb, _R)], xd_v)

            def do_row(r, alpha_acc):
                xcv0 = xc_v[r, head]
                xdv0 = xd_v[r, head]
                t_c = jnp.full((_L,), xcv0[0])
                t_d = jnp.full((_L,), xdv0[0])
                lo = t_c - t_d
                hi = t_c + t_d
                frac = jnp.minimum(1.0, (0.0 - lo) / ((hi - lo) + _EPS))
                a1 = jnp.where(hi <= 0.0, 1.0,
                               jnp.where(lo > 0.0, 0.0, frac))
                a2 = 1.0 - a1
                amax = jnp.maximum(a1, a2)
                ap1 = a1 / (amax + _EPS)
                ap2 = a2 / (amax + _EPS)
                sinv = 1.0 / (a1 + a2 + _EPS)
                u1 = a1 * sinv
                u2 = a2 * sinv
                p11 = ap1 + (1.0 - ap1) * u1
                p12 = (1.0 - ap1) * u2
                p21 = (1.0 - ap2) * u1
                p22 = ap2 + (1.0 - ap2) * u2

                upper_l = jnp.minimum(hi, 0.0)
                cL = (lo + upper_l) * 0.5
                dL = (upper_l - lo) * 0.5
                lower_r = jnp.maximum(lo, 0.0)
                cR = (lower_r + hi) * 0.5
                dR = (hi - lower_r) * 0.5
                c1_0 = cL * wb0 + bb0
                d1_0 = dL * awb0
                c2_0 = cR * wo0 + bo0
                nc1_0 = p11 * c1_0 + p12 * c2_0
                nc2_0 = p21 * c1_0 + p22 * c2_0
                nd1_0 = ap1 * d1_0
                nd2_0 = ap2 * c2_0
                nl0 = jnp.minimum(nc1_0 - nd1_0, nc2_0 - nd2_0)
                nr0 = jnp.maximum(nc1_0 + nd1_0, nc2_0 + nd2_0)
                oc0 = (nl0 + nr0) * 0.5
                od0 = (nr0 - nl0) * 0.5

                alpha = jnp.minimum(1.0, a1 + a2)
                alpha_acc = jnp.where(lane == (r % _L), alpha, alpha_acc)
                oa_v[pl.ds((r // _L) * _L, _L)] = alpha_acc

                for j in range(jvec):
                    wsl = pl.ds(j * _L, _L)
                    if j == 0:
                        xcv, xdv = xcv0, xdv0
                    else:
                        xcv = xc_v[r, wsl]
                        xdv = xd_v[r, wsl]
                    c1 = xcv * wb_v[wsl] + bb_v[wsl]
                    c2 = xcv * wo_v[wsl] + bo_v[wsl]
                    nd1 = xdv * awb_v[wsl] * ap1
                    nc1 = p11 * c1 + p12 * c2
                    nc2 = p21 * c1 + p22 * c2
                    nd2 = ap2 * c2
                    nl = jnp.minimum(nc1 - nd1, nc2 - nd2)
                    nr = jnp.maximum(nc1 + nd1, nc2 + nd2)
                    occ = (nl + nr) * 0.5
                    odd = (nr - nl) * 0.5
                    if j == 0:
                        occ = jnp.where(mask0, oc0, occ)
                        odd = jnp.where(mask0, od0, odd)
                    oc_v[r, wsl] = occ
                    od_v[r, wsl] = odd
                return alpha_acc

            lax.fori_loop(0, _R, do_row, jnp.zeros((_L,), jnp.float32))

            pltpu.sync_copy(oc_v, oc_hbm.at[pl.ds(cb, _R)])
            pltpu.sync_copy(od_v, od_hbm.at[pl.ds(cb, _R)])
            pltpu.sync_copy(oa_v, oa_hbm.at[pl.ds(cb, _R)])
            return carry

        lax.fori_loop(0, chunks, do_chunk, 0)

    return k(x_c, x_delta, wb, bb, wo, bo)


def kernel(x_c, x_delta, w_body, b_body, w_orelse, b_orelse):
    n, d = x_c.shape
    oc, od, oa = _sc_call(x_c, x_delta, w_body, b_body, w_orelse, b_orelse)
    return oc, od, oa.reshape(n, 1)


# speedup vs baseline: 1.2204x; 1.2204x over previous
import functools

import jax
import jax.numpy as jnp
from jax import lax
from jax.experimental import pallas as pl
from jax.experimental.pallas import tpu as pltpu
from jax.experimental.pallas import tpu_sc as plsc

_EPS = 1e-12
_L = 16
_NC = 2
_NS = 16
_NW = _NC * _NS
_R = 32


def _sc_call(x_c, x_delta, wb, bb, wo, bo):
    n, d = x_c.shape
    rows_per_w = n // _NW
    chunks = rows_per_w // _R
    pairs = chunks // 2
    jvec = d // _L

    mesh = plsc.VectorSubcoreMesh(core_axis_name="c", subcore_axis_name="s")

    @functools.partial(
        pl.kernel,
        mesh=mesh,
        out_type=[
            jax.ShapeDtypeStruct((n, d), jnp.float32),
            jax.ShapeDtypeStruct((n, d), jnp.float32),
            jax.ShapeDtypeStruct((n,), jnp.float32),
        ],
        scratch_types=[
            pltpu.VMEM((2, _R, d), jnp.float32),
            pltpu.VMEM((2, _R, d), jnp.float32),
            pltpu.VMEM((2, _R, d), jnp.float32),
            pltpu.VMEM((2, _R, d), jnp.float32),
            pltpu.VMEM((2, _R), jnp.float32),
            pltpu.VMEM((d,), jnp.float32),
            pltpu.VMEM((d,), jnp.float32),
            pltpu.VMEM((d,), jnp.float32),
            pltpu.VMEM((d,), jnp.float32),
            pltpu.VMEM((d,), jnp.float32),
            pltpu.SemaphoreType.DMA,
            pltpu.SemaphoreType.DMA,
            pltpu.SemaphoreType.DMA,
            pltpu.SemaphoreType.DMA,
        ],
    )
    def k(xc_hbm, xd_hbm, wb_hbm, bb_hbm, wo_hbm, bo_hbm,
          oc_hbm, od_hbm, oa_hbm,
          xc_v, xd_v, oc_v, od_v, oa_v,
          wb_v, bb_v, wo_v, bo_v, awb_v,
          in_s0, in_s1, out_s0, out_s1):
        wid = lax.axis_index("s") * _NC + lax.axis_index("c")
        base = wid * rows_per_w
        in_sems = (in_s0, in_s1)
        out_sems = (out_s0, out_s1)

        pltpu.sync_copy(wb_hbm, wb_v)
        pltpu.sync_copy(bb_hbm, bb_v)
        pltpu.sync_copy(wo_hbm, wo_v)
        pltpu.sync_copy(bo_hbm, bo_v)
        for j in range(jvec):
            sl = pl.ds(j * _L, _L)
            awb_v[sl] = jnp.abs(wb_v[sl])
        head = pl.ds(0, _L)
        wb0 = wb_v[head][0]
        bb0 = bb_v[head][0]
        wo0 = wo_v[head][0]
        bo0 = bo_v[head][0]
        awb0 = awb_v[head][0]
        lane = lax.iota(jnp.int32, _L)
        mask0 = lane == 0

        def start_in(s, c):
            cb = base + c * _R
            pltpu.async_copy(xc_hbm.at[pl.ds(cb, _R)], xc_v.at[s], in_sems[s])
            pltpu.async_copy(xd_hbm.at[pl.ds(cb, _R)], xd_v.at[s], in_sems[s])

        def wait_in(s, c):
            cb = base + c * _R
            pltpu.make_async_copy(
                xc_hbm.at[pl.ds(cb, _R)], xc_v.at[s], in_sems[s]).wait()
            pltpu.make_async_copy(
                xd_hbm.at[pl.ds(cb, _R)], xd_v.at[s], in_sems[s]).wait()

        def start_out(s, c):
            cb = base + c * _R
            pltpu.async_copy(oc_v.at[s], oc_hbm.at[pl.ds(cb, _R)], out_sems[s])
            pltpu.async_copy(od_v.at[s], od_hbm.at[pl.ds(cb, _R)], out_sems[s])
            pltpu.async_copy(oa_v.at[s], oa_hbm.at[pl.ds(cb, _R)], out_sems[s])

        def wait_out(s, c):
            cb = base + c * _R
            pltpu.make_async_copy(
                oc_v.at[s], oc_hbm.at[pl.ds(cb, _R)], out_sems[s]).wait()
            pltpu.make_async_copy(
                od_v.at[s], od_hbm.at[pl.ds(cb, _R)], out_sems[s]).wait()
            pltpu.make_async_copy(
                oa_v.at[s], oa_hbm.at[pl.ds(cb, _R)], out_sems[s]).wait()

        def row_coeffs(s, r):
            xcv0 = xc_v[s, r, head]
            xdv0 = xd_v[s, r, head]
            t_c = jnp.full((_L,), xcv0[0])
            t_d = jnp.full((_L,), xdv0[0])
            lo = t_c - t_d
            hi = t_c + t_d
            frac = jnp.minimum(1.0, (0.0 - lo) / ((hi - lo) + _EPS))
            a1 = jnp.where(hi <= 0.0, 1.0, jnp.where(lo > 0.0, 0.0, frac))
            a2 = 1.0 - a1
            amax = jnp.maximum(a1, a2)
            rcm = 1.0 / (amax + _EPS)
            ap1 = a1 * rcm
            ap2 = a2 * rcm
            sinv = 1.0 / (a1 + a2 + _EPS)
            u1 = a1 * sinv
            u2 = a2 * sinv
            p11 = ap1 + (1.0 - ap1) * u1
            p12 = (1.0 - ap1) * u2
            p21 = (1.0 - ap2) * u1
            p22 = ap2 + (1.0 - ap2) * u2

            upper_l = jnp.minimum(hi, 0.0)
            cL = (lo + upper_l) * 0.5
            dL = (upper_l - lo) * 0.5
            lower_r = jnp.maximum(lo, 0.0)
            cR = (lower_r + hi) * 0.5
            dR = (hi - lower_r) * 0.5
            c1_0 = cL * wb0 + bb0
            d1_0 = dL * awb0
            c2_0 = cR * wo0 + bo0
            nc1_0 = p11 * c1_0 + p12 * c2_0
            nc2_0 = p21 * c1_0 + p22 * c2_0
            nd1_0 = ap1 * d1_0
            nd2_0 = ap2 * c2_0
            nl0 = jnp.minimum(nc1_0 - nd1_0, nc2_0 - nd2_0)
            nr0 = jnp.maximum(nc1_0 + nd1_0, nc2_0 + nd2_0)
            oc0 = (nl0 + nr0) * 0.5
            od0 = (nr0 - nl0) * 0.5
            alpha = jnp.minimum(1.0, a1 + a2)
            return (p11, p12, p21, p22, ap1, ap2, oc0, od0, alpha)

        def compute_chunk(s):
            def do_pair(r2, alpha_acc):
                r0 = r2 * 2
                r1 = r0 + 1
                cf0 = row_coeffs(s, r0)
                cf1 = row_coeffs(s, r1)
                alpha_acc = jnp.where(lane == (r0 % _L), cf0[8], alpha_acc)
                alpha_acc = jnp.where(lane == (r1 % _L), cf1[8], alpha_acc)
                oa_v[s, pl.ds((r0 // _L) * _L, _L)] = alpha_acc
                for j in range(jvec):
                    wsl = pl.ds(j * _L, _L)
                    wbj = wb_v[wsl]
                    bbj = bb_v[wsl]
                    woj = wo_v[wsl]
                    boj = bo_v[wsl]
                    awbj = awb_v[wsl]
                    for r, cf in ((r0, cf0), (r1, cf1)):
                        p11, p12, p21, p22, ap1, ap2, oc0, od0, _ = cf
                        xcv = xc_v[s, r, wsl]
                        xdv = xd_v[s, r, wsl]
                        c1 = xcv * wbj + bbj
                        c2 = xcv * woj + boj
                        nd1 = xdv * awbj * ap1
                        nc1 = p11 * c1 + p12 * c2
                        nc2 = p21 * c1 + p22 * c2
                        nd2 = ap2 * c2
                        nl = jnp.minimum(nc1 - nd1, nc2 - nd2)
                        nr = jnp.maximum(nc1 + nd1, nc2 + nd2)
                        occ = (nl + nr) * 0.5
                        odd = (nr - nl) * 0.5
                        if j == 0:
                            occ = jnp.where(mask0, oc0, occ)
                            odd = jnp.where(mask0, od0, odd)
                        oc_v[s, r, wsl] = occ
                        od_v[s, r, wsl] = odd
                return alpha_acc

            lax.fori_loop(0, _R // 2, do_pair, jnp.zeros((_L,), jnp.float32))

        start_in(0, 0)
        start_in(1, 1)

        def do_pair_of_chunks(g2, carry):
            for s in (0, 1):
                c = g2 * 2 + s
                wait_in(s, c)

                @pl.when(g2 > 0)
                def _():
                    wait_out(s, c - 2)

                compute_chunk(s)
                start_out(s, c)

                @pl.when(c + 2 < chunks)
                def _():
                    start_in(s, c + 2)
            return carry

        lax.fori_loop(0, pairs, do_pair_of_chunks, 0)
        wait_out(0, chunks - 2)
        wait_out(1, chunks - 1)

    return k(x_c, x_delta, wb, bb, wo, bo)


def kernel(x_c, x_delta, w_body, b_body, w_orelse, b_orelse):
    n, d = x_c.shape
    oc, od, oa = _sc_call(x_c, x_delta, w_body, b_body, w_orelse, b_orelse)
    return oc, od, oa.reshape(n, 1)


# speedup vs baseline: 1.2825x; 1.0508x over previous
import functools

import jax
import jax.numpy as jnp
from jax import lax
from jax.experimental import pallas as pl
from jax.experimental.pallas import tpu as pltpu
from jax.experimental.pallas import tpu_sc as plsc

_EPS = 1e-12
_L = 16
_NC = 2
_NS = 16
_NW = _NC * _NS
_R = 32


def _sc_call(x_c, x_delta, wb, bb, wo, bo):
    n, d = x_c.shape
    rows_per_w = n // _NW
    chunks = rows_per_w // _R
    pairs = chunks // 2
    jvec = d // _L

    mesh = plsc.VectorSubcoreMesh(core_axis_name="c", subcore_axis_name="s")

    @functools.partial(
        pl.kernel,
        mesh=mesh,
        out_type=[
            jax.ShapeDtypeStruct((n, d), jnp.float32),
            jax.ShapeDtypeStruct((n, d), jnp.float32),
            jax.ShapeDtypeStruct((n,), jnp.float32),
        ],
        scratch_types=[
            pltpu.VMEM((2, _R, d), jnp.float32),
            pltpu.VMEM((2, _R, d), jnp.float32),
            pltpu.VMEM((2, _R, d), jnp.float32),
            pltpu.VMEM((2, _R, d), jnp.float32),
            pltpu.VMEM((2, _R), jnp.float32),
            pltpu.VMEM((d,), jnp.float32),
            pltpu.VMEM((d,), jnp.float32),
            pltpu.VMEM((d,), jnp.float32),
            pltpu.VMEM((d,), jnp.float32),
            pltpu.VMEM((d,), jnp.float32),
            pltpu.SemaphoreType.DMA,
            pltpu.SemaphoreType.DMA,
            pltpu.SemaphoreType.DMA,
            pltpu.SemaphoreType.DMA,
        ],
    )
    def k(xc_hbm, xd_hbm, wb_hbm, bb_hbm, wo_hbm, bo_hbm,
          oc_hbm, od_hbm, oa_hbm,
          xc_v, xd_v, oc_v, od_v, oa_v,
          wb_v, bb_v, wo_v, bo_v, awb_v,
          in_s0, in_s1, out_s0, out_s1):
        wid = lax.axis_index("s") * _NC + lax.axis_index("c")
        base = wid * rows_per_w
        in_sems = (in_s0, in_s1)
        out_sems = (out_s0, out_s1)

        pltpu.sync_copy(wb_hbm, wb_v)
        pltpu.sync_copy(bb_hbm, bb_v)
        pltpu.sync_copy(wo_hbm, wo_v)
        pltpu.sync_copy(bo_hbm, bo_v)
        for j in range(jvec):
            sl = pl.ds(j * _L, _L)
            awb_v[sl] = jnp.abs(wb_v[sl])
        head = pl.ds(0, _L)
        wb0 = wb_v[head][0]
        bb0 = bb_v[head][0]
        wo0 = wo_v[head][0]
        bo0 = bo_v[head][0]
        awb0 = awb_v[head][0]
        lane = lax.iota(jnp.int32, _L)
        mask0 = lane == 0

        def start_in(s, c):
            cb = base + c * _R
            pltpu.async_copy(xc_hbm.at[pl.ds(cb, _R)], xc_v.at[s], in_sems[s])
            pltpu.async_copy(xd_hbm.at[pl.ds(cb, _R)], xd_v.at[s], in_sems[s])

        def wait_in(s, c):
            cb = base + c * _R
            pltpu.make_async_copy(
                xc_hbm.at[pl.ds(cb, _R)], xc_v.at[s], in_sems[s]).wait()
            pltpu.make_async_copy(
                xd_hbm.at[pl.ds(cb, _R)], xd_v.at[s], in_sems[s]).wait()

        def start_out(s, c):
            cb = base + c * _R
            pltpu.async_copy(oc_v.at[s], oc_hbm.at[pl.ds(cb, _R)], out_sems[s])
            pltpu.async_copy(od_v.at[s], od_hbm.at[pl.ds(cb, _R)], out_sems[s])
            pltpu.async_copy(oa_v.at[s], oa_hbm.at[pl.ds(cb, _R)], out_sems[s])

        def wait_out(s, c):
            cb = base + c * _R
            pltpu.make_async_copy(
                oc_v.at[s], oc_hbm.at[pl.ds(cb, _R)], out_sems[s]).wait()
            pltpu.make_async_copy(
                od_v.at[s], od_hbm.at[pl.ds(cb, _R)], out_sems[s]).wait()
            pltpu.make_async_copy(
                oa_v.at[s], oa_hbm.at[pl.ds(cb, _R)], out_sems[s]).wait()

        def row_coeffs(s, r):
            xcv0 = xc_v[s, r, head]
            xdv0 = xd_v[s, r, head]
            t_c = jnp.full((_L,), xcv0[0])
            t_d = jnp.full((_L,), xdv0[0])
            lo = t_c - t_d
            hi = t_c + t_d
            frac = jnp.minimum(1.0, (0.0 - lo) / ((hi - lo) + _EPS))
            a1 = jnp.where(hi <= 0.0, 1.0, jnp.where(lo > 0.0, 0.0, frac))
            a2 = 1.0 - a1
            amax = jnp.maximum(a1, a2)
            rcm = 1.0 / (amax + _EPS)
            ap1 = a1 * rcm
            ap2 = a2 * rcm
            sinv = 1.0 / (a1 + a2 + _EPS)
            u1 = a1 * sinv
            u2 = a2 * sinv
            q11 = (ap1 + (1.0 - ap1) * u1) * 0.5
            q12 = ((1.0 - ap1) * u2) * 0.5
            q21 = ((1.0 - ap2) * u1) * 0.5
            q22 = (ap2 + (1.0 - ap2) * u2) * 0.5
            ap1h = ap1 * 0.5
            ap2h = ap2 * 0.5

            upper_l = jnp.minimum(hi, 0.0)
            cL = (lo + upper_l) * 0.5
            dL = (upper_l - lo) * 0.5
            lower_r = jnp.maximum(lo, 0.0)
            cR = (lower_r + hi) * 0.5
            c1_0 = cL * wb0 + bb0
            d1_0 = dL * awb0
            c2_0 = cR * wo0 + bo0
            hnc1_0 = q11 * c1_0 + q12 * c2_0
            hnc2_0 = q21 * c1_0 + q22 * c2_0
            hnd1_0 = ap1h * d1_0
            hnd2_0 = ap2h * c2_0
            hnl0 = jnp.minimum(hnc1_0 - hnd1_0, hnc2_0 - hnd2_0)
            hnr0 = jnp.maximum(hnc1_0 + hnd1_0, hnc2_0 + hnd2_0)
            oc0 = hnl0 + hnr0
            od0 = hnr0 - hnl0
            alpha = jnp.minimum(1.0, a1 + a2)
            return (q11, q12, q21, q22, ap1h, ap2h, oc0, od0, alpha)

        def compute_chunk(s):
            z = jnp.zeros((_L,), jnp.float32)

            def do_row(r, accs):
                acc0, acc1 = accs
                q11, q12, q21, q22, ap1h, ap2h, oc0, od0, alpha = \
                    row_coeffs(s, r)
                acc0 = jnp.where(lane == r, alpha, acc0)
                acc1 = jnp.where(lane == (r - _L), alpha, acc1)
                for j in range(jvec):
                    wsl = pl.ds(j * _L, _L)
                    xcv = xc_v[s, r, wsl]
                    xdv = xd_v[s, r, wsl]
                    c1 = xcv * wb_v[wsl] + bb_v[wsl]
                    c2 = xcv * wo_v[wsl] + bo_v[wsl]
                    hnd1 = xdv * awb_v[wsl] * ap1h
                    hnc1 = q11 * c1 + q12 * c2
                    hnc2 = q21 * c1 + q22 * c2
                    hnd2 = ap2h * c2
                    hnl = jnp.minimum(hnc1 - hnd1, hnc2 - hnd2)
                    hnr = jnp.maximum(hnc1 + hnd1, hnc2 + hnd2)
                    occ = hnl + hnr
                    odd = hnr - hnl
                    if j == 0:
                        occ = jnp.where(mask0, oc0, occ)
                        odd = jnp.where(mask0, od0, odd)
                    oc_v[s, r, wsl] = occ
                    od_v[s, r, wsl] = odd
                return (acc0, acc1)

            acc0, acc1 = plsc.parallel_loop(
                0, _R, unroll=2, carry=(z, z))(do_row)
            oa_v[s, pl.ds(0, _L)] = acc0
            oa_v[s, pl.ds(_L, _L)] = acc1

        start_in(0, 0)
        start_in(1, 1)

        def do_pair_of_chunks(g2, carry):
            for s in (0, 1):
                c = g2 * 2 + s
                wait_in(s, c)

                @pl.when(g2 > 0)
                def _():
                    wait_out(s, c - 2)

                compute_chunk(s)
                start_out(s, c)

                @pl.when(c + 2 < chunks)
                def _():
                    start_in(s, c + 2)
            return carry

        lax.fori_loop(0, pairs, do_pair_of_chunks, 0)
        wait_out(0, chunks - 2)
        wait_out(1, chunks - 1)

    return k(x_c, x_delta, wb, bb, wo, bo)


def kernel(x_c, x_delta, w_body, b_body, w_orelse, b_orelse):
    n, d = x_c.shape
    oc, od, oa = _sc_call(x_c, x_delta, w_body, b_body, w_orelse, b_orelse)
    return oc, od, oa.reshape(n, 1)


# speedup vs baseline: 1.2912x; 1.0068x over previous
import functools

import jax
import jax.numpy as jnp
from jax import lax
from jax.experimental import pallas as pl
from jax.experimental.pallas import tpu as pltpu
from jax.experimental.pallas import tpu_sc as plsc

_EPS = 1e-12
_L = 16
_NC = 2
_NS = 16
_NW = _NC * _NS
_R = 32


def _sc_call(x_c, x_delta, wb, bb, wo, bo):
    n, d = x_c.shape
    rows_per_w = n // _NW
    chunks = rows_per_w // _R
    pairs = chunks // 2
    jvec = d // _L

    mesh = plsc.VectorSubcoreMesh(core_axis_name="c", subcore_axis_name="s")

    @functools.partial(
        pl.kernel,
        mesh=mesh,
        out_type=[
            jax.ShapeDtypeStruct((n, d), jnp.float32),
            jax.ShapeDtypeStruct((n, d), jnp.float32),
            jax.ShapeDtypeStruct((n,), jnp.float32),
        ],
        scratch_types=[
            pltpu.VMEM((2, _R, d), jnp.float32),
            pltpu.VMEM((2, _R, d), jnp.float32),
            pltpu.VMEM((2, _R, d), jnp.float32),
            pltpu.VMEM((2, _R, d), jnp.float32),
            pltpu.VMEM((2, _R), jnp.float32),
            pltpu.VMEM((d,), jnp.float32),
            pltpu.VMEM((d,), jnp.float32),
            pltpu.VMEM((d,), jnp.float32),
            pltpu.VMEM((d,), jnp.float32),
            pltpu.VMEM((d,), jnp.float32),
            pltpu.SemaphoreType.DMA,
            pltpu.SemaphoreType.DMA,
            pltpu.SemaphoreType.DMA,
            pltpu.SemaphoreType.DMA,
        ],
    )
    def k(xc_hbm, xd_hbm, wb_hbm, bb_hbm, wo_hbm, bo_hbm,
          oc_hbm, od_hbm, oa_hbm,
          xc_v, xd_v, oc_v, od_v, oa_v,
          wb_v, bb_v, wo_v, bo_v, awb_v,
          in_s0, in_s1, out_s0, out_s1):
        wid = lax.axis_index("s") * _NC + lax.axis_index("c")
        base = wid * rows_per_w
        in_sems = (in_s0, in_s1)
        out_sems = (out_s0, out_s1)

        pltpu.sync_copy(wb_hbm, wb_v)
        pltpu.sync_copy(bb_hbm, bb_v)
        pltpu.sync_copy(wo_hbm, wo_v)
        pltpu.sync_copy(bo_hbm, bo_v)
        for j in range(jvec):
            sl = pl.ds(j * _L, _L)
            awb_v[sl] = jnp.abs(wb_v[sl])
        head = pl.ds(0, _L)
        wb0 = wb_v[head][0]
        bb0 = bb_v[head][0]
        wo0 = wo_v[head][0]
        bo0 = bo_v[head][0]
        awb0 = awb_v[head][0]
        lane = lax.iota(jnp.int32, _L)
        mask0 = lane == 0

        def start_in(s, c):
            cb = base + c * _R
            pltpu.async_copy(xc_hbm.at[pl.ds(cb, _R)], xc_v.at[s], in_sems[s])
            pltpu.async_copy(xd_hbm.at[pl.ds(cb, _R)], xd_v.at[s], in_sems[s])

        def wait_in(s, c):
            cb = base + c * _R
            pltpu.make_async_copy(
                xc_hbm.at[pl.ds(cb, _R)], xc_v.at[s], in_sems[s]).wait()
            pltpu.make_async_copy(
                xd_hbm.at[pl.ds(cb, _R)], xd_v.at[s], in_sems[s]).wait()

        def start_out(s, c):
            cb = base + c * _R
            pltpu.async_copy(oc_v.at[s], oc_hbm.at[pl.ds(cb, _R)], out_sems[s])
            pltpu.async_copy(od_v.at[s], od_hbm.at[pl.ds(cb, _R)], out_sems[s])
            pltpu.async_copy(oa_v.at[s], oa_hbm.at[pl.ds(cb, _R)], out_sems[s])

        def wait_out(s, c):
            cb = base + c * _R
            pltpu.make_async_copy(
                oc_v.at[s], oc_hbm.at[pl.ds(cb, _R)], out_sems[s]).wait()
            pltpu.make_async_copy(
                od_v.at[s], od_hbm.at[pl.ds(cb, _R)], out_sems[s]).wait()
            pltpu.make_async_copy(
                oa_v.at[s], oa_hbm.at[pl.ds(cb, _R)], out_sems[s]).wait()

        def row_coeffs(s, r):
            xcv0 = xc_v[s, r, head]
            xdv0 = xd_v[s, r, head]
            t_c = jnp.full((_L,), xcv0[0])
            t_d = jnp.full((_L,), xdv0[0])
            lo = t_c - t_d
            hi = t_c + t_d
            frac = jnp.minimum(1.0, (0.0 - lo) / ((hi - lo) + _EPS))
            a1 = jnp.where(hi <= 0.0, 1.0, jnp.where(lo > 0.0, 0.0, frac))
            a2 = 1.0 - a1
            amax = jnp.maximum(a1, a2)
            rcm = 1.0 / (amax + _EPS)
            ap1 = a1 * rcm
            ap2 = a2 * rcm
            sinv = 1.0 / (a1 + a2 + _EPS)
            u1 = a1 * sinv
            q11 = (ap1 + (1.0 - ap1) * u1) * 0.5
            q21 = ((1.0 - ap2) * u1) * 0.5
            ap1h = ap1 * 0.5
            ap2h = ap2 * 0.5

            upper_l = jnp.minimum(hi, 0.0)
            cL = (lo + upper_l) * 0.5
            dL = (upper_l - lo) * 0.5
            lower_r = jnp.maximum(lo, 0.0)
            cR = (lower_r + hi) * 0.5
            c1_0 = cL * wb0 + bb0
            d1_0 = dL * awb0
            c2_0 = cR * wo0 + bo0
            e0 = c1_0 - c2_0
            g1_0 = ap1h * d1_0
            g2_0 = ap2h * c2_0
            mn0 = jnp.minimum(q11 * e0 - g1_0, q21 * e0 - g2_0)
            mx0 = jnp.maximum(q11 * e0 + g1_0, q21 * e0 + g2_0)
            oc0 = c2_0 + mn0 + mx0
            od0 = mx0 - mn0
            alpha = jnp.minimum(1.0, a1 + a2)
            return (q11, q21, ap1h, ap2h, oc0, od0, alpha)

        def compute_chunk(s):
            z = jnp.zeros((_L,), jnp.float32)

            def do_row(r, accs):
                acc0, acc1 = accs
                q11, q21, ap1h, ap2h, oc0, od0, alpha = row_coeffs(s, r)
                acc0 = jnp.where(lane == r, alpha, acc0)
                acc1 = jnp.where(lane == (r - _L), alpha, acc1)
                for j in range(jvec):
                    wsl = pl.ds(j * _L, _L)
                    xcv = xc_v[s, r, wsl]
                    xdv = xd_v[s, r, wsl]
                    c1 = xcv * wb_v[wsl] + bb_v[wsl]
                    c2 = xcv * wo_v[wsl] + bo_v[wsl]
                    e = c1 - c2
                    g1 = xdv * awb_v[wsl] * ap1h
                    g2 = ap2h * c2
                    t1 = q11 * e
                    t2 = q21 * e
                    mn = jnp.minimum(t1 - g1, t2 - g2)
                    mx = jnp.maximum(t1 + g1, t2 + g2)
                    occ = c2 + mn + mx
                    odd = mx - mn
                    if j == 0:
                        occ = jnp.where(mask0, oc0, occ)
                        odd = jnp.where(mask0, od0, odd)
                    oc_v[s, r, wsl] = occ
                    od_v[s, r, wsl] = odd
                return (acc0, acc1)

            acc0, acc1 = plsc.parallel_loop(
                0, _R, unroll=2, carry=(z, z))(do_row)
            oa_v[s, pl.ds(0, _L)] = acc0
            oa_v[s, pl.ds(_L, _L)] = acc1

        start_in(0, 0)
        start_in(1, 1)

        def do_pair_of_chunks(g2, carry):
            for s in (0, 1):
                c = g2 * 2 + s
                wait_in(s, c)

                @pl.when(g2 > 0)
                def _():
                    wait_out(s, c - 2)

                compute_chunk(s)
                start_out(s, c)

                @pl.when(c + 2 < chunks)
                def _():
                    start_in(s, c + 2)
            return carry

        lax.fori_loop(0, pairs, do_pair_of_chunks, 0)
        wait_out(0, chunks - 2)
        wait_out(1, chunks - 1)

    return k(x_c, x_delta, wb, bb, wo, bo)


def kernel(x_c, x_delta, w_body, b_body, w_orelse, b_orelse):
    n, d = x_c.shape
    oc, od, oa = _sc_call(x_c, x_delta, w_body, b_body, w_orelse, b_orelse)
    return oc, od, oa.reshape(n, 1)


# speedup vs baseline: 2.2747x; 1.7617x over previous
import functools

import jax
import jax.numpy as jnp
from jax import lax
from jax.experimental import pallas as pl
from jax.experimental.pallas import tpu as pltpu
from jax.experimental.pallas import tpu_sc as plsc

_EPS = 1e-12
_L = 16
_NC = 2
_NS = 16
_NW = _NC * _NS
_R = 32


def _sc_call(x_c, x_delta, wb, bb, wo, bo):
    n, d = x_c.shape
    rows_per_w = n // _NW
    chunks = rows_per_w // _R
    pairs = chunks // 2
    jvec = d // _L

    mesh = plsc.VectorSubcoreMesh(core_axis_name="c", subcore_axis_name="s")

    @functools.partial(
        pl.kernel,
        mesh=mesh,
        out_type=[
            jax.ShapeDtypeStruct((n, d), jnp.float32),
            jax.ShapeDtypeStruct((n, d), jnp.float32),
            jax.ShapeDtypeStruct((n,), jnp.float32),
        ],
        scratch_types=[
            pltpu.VMEM((2, _R, d), jnp.float32),
            pltpu.VMEM((2, _R, d), jnp.float32),
            pltpu.VMEM((2, _R, d), jnp.float32),
            pltpu.VMEM((2, _R, d), jnp.float32),
            pltpu.VMEM((2, _R), jnp.float32),
            pltpu.VMEM((d,), jnp.float32),
            pltpu.VMEM((d,), jnp.float32),
            pltpu.VMEM((d,), jnp.float32),
            pltpu.VMEM((d,), jnp.float32),
            pltpu.VMEM((d,), jnp.float32),
            pltpu.SemaphoreType.DMA,
            pltpu.SemaphoreType.DMA,
            pltpu.SemaphoreType.DMA,
            pltpu.SemaphoreType.DMA,
        ],
    )
    def k(xc_hbm, xd_hbm, wb_hbm, bb_hbm, wo_hbm, bo_hbm,
          oc_hbm, od_hbm, oa_hbm,
          xc_v, xd_v, oc_v, od_v, oa_v,
          wb_v, bb_v, wo_v, bo_v, awb_v,
          in_s0, in_s1, out_s0, out_s1):
        wid = lax.axis_index("s") * _NC + lax.axis_index("c")
        base = wid * rows_per_w
        in_sems = (in_s0, in_s1)
        out_sems = (out_s0, out_s1)

        pltpu.sync_copy(wb_hbm, wb_v)
        pltpu.sync_copy(bb_hbm, bb_v)
        pltpu.sync_copy(wo_hbm, wo_v)
        pltpu.sync_copy(bo_hbm, bo_v)
        for j in range(jvec):
            sl = pl.ds(j * _L, _L)
            awb_v[sl] = jnp.abs(wb_v[sl])
        head = pl.ds(0, _L)
        wb0 = wb_v[head][0]
        bb0 = bb_v[head][0]
        wo0 = wo_v[head][0]
        bo0 = bo_v[head][0]
        awb0 = awb_v[head][0]
        lane = lax.iota(jnp.int32, _L)
        mask0 = lane == 0

        def start_in(s, c):
            cb = base + c * _R
            pltpu.async_copy(xc_hbm.at[pl.ds(cb, _R)], xc_v.at[s], in_sems[s])
            pltpu.async_copy(xd_hbm.at[pl.ds(cb, _R)], xd_v.at[s], in_sems[s])

        def wait_in(s, c):
            cb = base + c * _R
            pltpu.make_async_copy(
                xc_hbm.at[pl.ds(cb, _R)], xc_v.at[s], in_sems[s]).wait()
            pltpu.make_async_copy(
                xd_hbm.at[pl.ds(cb, _R)], xd_v.at[s], in_sems[s]).wait()

        def start_out(s, c):
            cb = base + c * _R
            pltpu.async_copy(oc_v.at[s], oc_hbm.at[pl.ds(cb, _R)], out_sems[s])
            pltpu.async_copy(od_v.at[s], od_hbm.at[pl.ds(cb, _R)], out_sems[s])
            pltpu.async_copy(oa_v.at[s], oa_hbm.at[pl.ds(cb, _R)], out_sems[s])

        def wait_out(s, c):
            cb = base + c * _R
            pltpu.make_async_copy(
                oc_v.at[s], oc_hbm.at[pl.ds(cb, _R)], out_sems[s]).wait()
            pltpu.make_async_copy(
                od_v.at[s], od_hbm.at[pl.ds(cb, _R)], out_sems[s]).wait()
            pltpu.make_async_copy(
                oa_v.at[s], oa_hbm.at[pl.ds(cb, _R)], out_sems[s]).wait()

        def row_coeffs(s, r):
            xcv0 = xc_v[s, r, head]
            xdv0 = xd_v[s, r, head]
            t_c = jnp.full((_L,), xcv0[0])
            t_d = jnp.full((_L,), xdv0[0])
            lo = t_c - t_d
            hi = t_c + t_d
            frac = jnp.minimum(1.0, (0.0 - lo) / ((hi - lo) + _EPS))
            a1 = jnp.where(hi <= 0.0, 1.0, jnp.where(lo > 0.0, 0.0, frac))
            a2 = 1.0 - a1
            amax = jnp.maximum(a1, a2)
            rcm = 1.0 / (amax + _EPS)
            ap1 = a1 * rcm
            ap2 = a2 * rcm
            sinv = 1.0 / (a1 + a2 + _EPS)
            u1 = a1 * sinv
            q11 = (ap1 + (1.0 - ap1) * u1) * 0.5
            q21 = ((1.0 - ap2) * u1) * 0.5
            ap1h = ap1 * 0.5
            ap2h = ap2 * 0.5

            upper_l = jnp.minimum(hi, 0.0)
            cL = (lo + upper_l) * 0.5
            dL = (upper_l - lo) * 0.5
            lower_r = jnp.maximum(lo, 0.0)
            cR = (lower_r + hi) * 0.5
            c1_0 = cL * wb0 + bb0
            d1_0 = dL * awb0
            c2_0 = cR * wo0 + bo0
            e0 = c1_0 - c2_0
            g1_0 = ap1h * d1_0
            g2_0 = ap2h * c2_0
            mn0 = jnp.minimum(q11 * e0 - g1_0, q21 * e0 - g2_0)
            mx0 = jnp.maximum(q11 * e0 + g1_0, q21 * e0 + g2_0)
            oc0 = c2_0 + mn0 + mx0
            od0 = mx0 - mn0
            alpha = jnp.minimum(1.0, a1 + a2)
            return (q11, q21, ap1h, ap2h, oc0, od0, alpha)

        def compute_chunk(s):
            z = jnp.zeros((_L,), jnp.float32)

            def do_row(r, accs):
                acc0, acc1 = accs
                q11, q21, ap1h, ap2h, oc0, od0, alpha = row_coeffs(s, r)
                acc0 = jnp.where(lane == r, alpha, acc0)
                acc1 = jnp.where(lane == (r - _L), alpha, acc1)
                gb = 4
                for b in range(jvec // gb):
                    ins = []
                    for q in range(gb):
                        wsl = pl.ds((b * gb + q) * _L, _L)
                        ins.append((wsl, xc_v[s, r, wsl], xd_v[s, r, wsl],
                                    wb_v[wsl], bb_v[wsl], wo_v[wsl],
                                    bo_v[wsl], awb_v[wsl]))
                    outs = []
                    for q, (wsl, xcv, xdv, wbj, bbj, woj, boj, awbj) \
                            in enumerate(ins):
                        c1 = xcv * wbj + bbj
                        c2 = xcv * woj + boj
                        e = c1 - c2
                        g1 = xdv * awbj * ap1h
                        g2 = ap2h * c2
                        t1 = q11 * e
                        t2 = q21 * e
                        mn = jnp.minimum(t1 - g1, t2 - g2)
                        mx = jnp.maximum(t1 + g1, t2 + g2)
                        occ = c2 + mn + mx
                        odd = mx - mn
                        if b == 0 and q == 0:
                            occ = jnp.where(mask0, oc0, occ)
                            odd = jnp.where(mask0, od0, odd)
                        outs.append((wsl, occ, odd))
                    for wsl, occ, odd in outs:
                        oc_v[s, r, wsl] = occ
                        od_v[s, r, wsl] = odd
                return (acc0, acc1)

            acc0, acc1 = plsc.parallel_loop(
                0, _R, unroll=2, carry=(z, z))(do_row)
            oa_v[s, pl.ds(0, _L)] = acc0
            oa_v[s, pl.ds(_L, _L)] = acc1

        start_in(0, 0)
        start_in(1, 1)

        def do_pair_of_chunks(g2, carry):
            for s in (0, 1):
                c = g2 * 2 + s
                wait_in(s, c)

                @pl.when(g2 > 0)
                def _():
                    wait_out(s, c - 2)

                compute_chunk(s)
                start_out(s, c)

                @pl.when(c + 2 < chunks)
                def _():
                    start_in(s, c + 2)
            return carry

        lax.fori_loop(0, pairs, do_pair_of_chunks, 0)
        wait_out(0, chunks - 2)
        wait_out(1, chunks - 1)

    return k(x_c, x_delta, wb, bb, wo, bo)


def kernel(x_c, x_delta, w_body, b_body, w_orelse, b_orelse):
    n, d = x_c.shape
    oc, od, oa = _sc_call(x_c, x_delta, w_body, b_body, w_orelse, b_orelse)
    return oc, od, oa.reshape(n, 1)


# speedup vs baseline: 2.8818x; 1.2669x over previous
import functools

import jax
import jax.numpy as jnp
from jax import lax
from jax.experimental import pallas as pl
from jax.experimental.pallas import tpu as pltpu
from jax.experimental.pallas import tpu_sc as plsc

_EPS = 1e-12
_L = 16
_NC = 2
_NS = 16
_NW = _NC * _NS
_R = 32


def _sc_call(x_c, x_delta, wb, bb, wo, bo):
    n, d = x_c.shape
    rows_per_w = n // _NW
    chunks = rows_per_w // _R
    pairs = chunks // 2
    jvec = d // _L

    mesh = plsc.VectorSubcoreMesh(core_axis_name="c", subcore_axis_name="s")

    @functools.partial(
        pl.kernel,
        mesh=mesh,
        out_type=[
            jax.ShapeDtypeStruct((n, d), jnp.float32),
            jax.ShapeDtypeStruct((n, d), jnp.float32),
            jax.ShapeDtypeStruct((n,), jnp.float32),
        ],
        scratch_types=[
            pltpu.VMEM((2, _R, d), jnp.float32),
            pltpu.VMEM((2, _R, d), jnp.float32),
            pltpu.VMEM((2, _R, d), jnp.float32),
            pltpu.VMEM((2, _R, d), jnp.float32),
            pltpu.VMEM((2, _R), jnp.float32),
            pltpu.VMEM((d,), jnp.float32),
            pltpu.VMEM((d,), jnp.float32),
            pltpu.VMEM((d,), jnp.float32),
            pltpu.VMEM((d,), jnp.float32),
            pltpu.VMEM((d,), jnp.float32),
            pltpu.SemaphoreType.DMA,
            pltpu.SemaphoreType.DMA,
            pltpu.SemaphoreType.DMA,
            pltpu.SemaphoreType.DMA,
        ],
    )
    def k(xc_hbm, xd_hbm, wb_hbm, bb_hbm, wo_hbm, bo_hbm,
          oc_hbm, od_hbm, oa_hbm,
          xc_v, xd_v, oc_v, od_v, oa_v,
          wb_v, bb_v, wo_v, bo_v, awb_v,
          in_s0, in_s1, out_s0, out_s1):
        wid = lax.axis_index("s") * _NC + lax.axis_index("c")
        base = wid * rows_per_w
        in_sems = (in_s0, in_s1)
        out_sems = (out_s0, out_s1)

        pltpu.sync_copy(wb_hbm, wb_v)
        pltpu.sync_copy(bb_hbm, bb_v)
        pltpu.sync_copy(wo_hbm, wo_v)
        pltpu.sync_copy(bo_hbm, bo_v)
        for j in range(jvec):
            sl = pl.ds(j * _L, _L)
            awb_v[sl] = jnp.abs(wb_v[sl])
        head = pl.ds(0, _L)
        wb0 = wb_v[head][0]
        bb0 = bb_v[head][0]
        wo0 = wo_v[head][0]
        bo0 = bo_v[head][0]
        awb0 = awb_v[head][0]
        lane = lax.iota(jnp.int32, _L)
        mask0 = lane == 0

        def start_in(s, c):
            cb = base + c * _R
            pltpu.async_copy(xc_hbm.at[pl.ds(cb, _R)], xc_v.at[s], in_sems[s])
            pltpu.async_copy(xd_hbm.at[pl.ds(cb, _R)], xd_v.at[s], in_sems[s])

        def wait_in(s, c):
            cb = base + c * _R
            pltpu.make_async_copy(
                xc_hbm.at[pl.ds(cb, _R)], xc_v.at[s], in_sems[s]).wait()
            pltpu.make_async_copy(
                xd_hbm.at[pl.ds(cb, _R)], xd_v.at[s], in_sems[s]).wait()

        def start_out(s, c):
            cb = base + c * _R
            pltpu.async_copy(oc_v.at[s], oc_hbm.at[pl.ds(cb, _R)], out_sems[s])
            pltpu.async_copy(od_v.at[s], od_hbm.at[pl.ds(cb, _R)], out_sems[s])
            pltpu.async_copy(oa_v.at[s], oa_hbm.at[pl.ds(cb, _R)], out_sems[s])

        def wait_out(s, c):
            cb = base + c * _R
            pltpu.make_async_copy(
                oc_v.at[s], oc_hbm.at[pl.ds(cb, _R)], out_sems[s]).wait()
            pltpu.make_async_copy(
                od_v.at[s], od_hbm.at[pl.ds(cb, _R)], out_sems[s]).wait()
            pltpu.make_async_copy(
                oa_v.at[s], oa_hbm.at[pl.ds(cb, _R)], out_sems[s]).wait()

        def row_coeffs(s, r):
            xcv0 = xc_v[s, r, head]
            xdv0 = xd_v[s, r, head]
            t_c = jnp.full((_L,), xcv0[0])
            t_d = jnp.full((_L,), xdv0[0])
            lo = t_c - t_d
            hi = t_c + t_d
            frac = jnp.minimum(1.0, (0.0 - lo) / ((hi - lo) + _EPS))
            a1 = jnp.where(hi <= 0.0, 1.0, jnp.where(lo > 0.0, 0.0, frac))
            a2 = 1.0 - a1
            amax = jnp.maximum(a1, a2)
            rcm = 1.0 / (amax + _EPS)
            ap1 = a1 * rcm
            ap2 = a2 * rcm
            sinv = 1.0 / (a1 + a2 + _EPS)
            u1 = a1 * sinv
            q11 = (ap1 + (1.0 - ap1) * u1) * 0.5
            q21 = ((1.0 - ap2) * u1) * 0.5
            ap1h = ap1 * 0.5
            ap2h = ap2 * 0.5

            upper_l = jnp.minimum(hi, 0.0)
            cL = (lo + upper_l) * 0.5
            dL = (upper_l - lo) * 0.5
            lower_r = jnp.maximum(lo, 0.0)
            cR = (lower_r + hi) * 0.5
            c1_0 = cL * wb0 + bb0
            d1_0 = dL * awb0
            c2_0 = cR * wo0 + bo0
            e0 = c1_0 - c2_0
            g1_0 = ap1h * d1_0
            g2_0 = ap2h * c2_0
            mn0 = jnp.minimum(q11 * e0 - g1_0, q21 * e0 - g2_0)
            mx0 = jnp.maximum(q11 * e0 + g1_0, q21 * e0 + g2_0)
            oc0 = c2_0 + mn0 + mx0
            od0 = mx0 - mn0
            alpha = jnp.minimum(1.0, a1 + a2)
            return (q11, q21, ap1h, ap2h, oc0, od0, alpha)

        def compute_chunk(s):
            z = jnp.zeros((_L,), jnp.float32)

            def do_row(r, accs):
                acc0, acc1 = accs
                q11, q21, ap1h, ap2h, oc0, od0, alpha = row_coeffs(s, r)
                acc0 = jnp.where(lane == r, alpha, acc0)
                acc1 = jnp.where(lane == (r - _L), alpha, acc1)
                gb = 16
                for b in range(jvec // gb):
                    ins = []
                    for q in range(gb):
                        wsl = pl.ds((b * gb + q) * _L, _L)
                        ins.append((wsl, xc_v[s, r, wsl], xd_v[s, r, wsl],
                                    wb_v[wsl], bb_v[wsl], wo_v[wsl],
                                    bo_v[wsl], awb_v[wsl]))
                    outs = []
                    for q, (wsl, xcv, xdv, wbj, bbj, woj, boj, awbj) \
                            in enumerate(ins):
                        c1 = xcv * wbj + bbj
                        c2 = xcv * woj + boj
                        e = c1 - c2
                        g1 = xdv * awbj * ap1h
                        g2 = ap2h * c2
                        t1 = q11 * e
                        t2 = q21 * e
                        mn = jnp.minimum(t1 - g1, t2 - g2)
                        mx = jnp.maximum(t1 + g1, t2 + g2)
                        occ = c2 + mn + mx
                        odd = mx - mn
                        if b == 0 and q == 0:
                            occ = jnp.where(mask0, oc0, occ)
                            odd = jnp.where(mask0, od0, odd)
                        outs.append((wsl, occ, odd))
                    for wsl, occ, odd in outs:
                        oc_v[s, r, wsl] = occ
                        od_v[s, r, wsl] = odd
                return (acc0, acc1)

            acc0, acc1 = plsc.parallel_loop(
                0, _R, unroll=2, carry=(z, z))(do_row)
            oa_v[s, pl.ds(0, _L)] = acc0
            oa_v[s, pl.ds(_L, _L)] = acc1

        start_in(0, 0)
        start_in(1, 1)

        def do_pair_of_chunks(g2, carry):
            for s in (0, 1):
                c = g2 * 2 + s
                wait_in(s, c)

                @pl.when(g2 > 0)
                def _():
                    wait_out(s, c - 2)

                compute_chunk(s)
                start_out(s, c)

                @pl.when(c + 2 < chunks)
                def _():
                    start_in(s, c + 2)
            return carry

        lax.fori_loop(0, pairs, do_pair_of_chunks, 0)
        wait_out(0, chunks - 2)
        wait_out(1, chunks - 1)

    return k(x_c, x_delta, wb, bb, wo, bo)


def kernel(x_c, x_delta, w_body, b_body, w_orelse, b_orelse):
    n, d = x_c.shape
    oc, od, oa = _sc_call(x_c, x_delta, w_body, b_body, w_orelse, b_orelse)
    return oc, od, oa.reshape(n, 1)
